# Initial kernel scaffold; baseline (speedup 1.0000x reference)
import jax
import jax.numpy as jnp
from jax.experimental import pallas as pl


def kernel(verts, target_points, target_normals):
    raise NotImplementedError("write your pallas kernel here")


# speedup vs baseline: 1.9671x; 1.9671x over previous
import jax
import jax.numpy as jnp
from jax.experimental import pallas as pl
from jax.experimental.pallas import tpu as pltpu

_ALPHA = 0.3
_INV2A2 = 1.0 / (2.0 * _ALPHA * _ALPHA)


def _chamfer_kernel(x_ref, y_ref, out_ref, cmin_ref, acc_ref):
    b = pl.program_id(0)
    i = pl.program_id(1)
    nb = pl.num_programs(0)
    ni = pl.num_programs(1)

    @pl.when((b == 0) & (i == 0))
    def _init_acc():
        acc_ref[0, 0] = 0.0

    @pl.when(i == 0)
    def _init_cmin():
        cmin_ref[...] = jnp.full_like(cmin_ref, jnp.inf)

    x = x_ref[0]
    y = y_ref[0]
    xy = jnp.dot(x, y, preferred_element_type=jnp.float32)
    x2 = jnp.sum(x * x, axis=1, keepdims=True)
    y2 = jnp.sum(y * y, axis=0, keepdims=True)
    d = (x2 + y2) - 2.0 * xy

    rmin = jnp.min(d, axis=1)
    acc_ref[0, 0] += jnp.sum(jnp.exp(-(rmin * rmin) * _INV2A2) * rmin)

    cmin_ref[...] = jnp.minimum(cmin_ref[...], jnp.min(d, axis=0, keepdims=True))

    @pl.when(i == ni - 1)
    def _batch_end():
        c = cmin_ref[0]
        acc_ref[0, 0] += jnp.sum(jnp.exp(-(c * c) * _INV2A2) * c)

    @pl.when((b == nb - 1) & (i == ni - 1))
    def _final():
        out_ref[0, 0] = acc_ref[0, 0] / nb


def _chamfer_pallas(xp, yp):
    B, N, _ = xp.shape
    M = yp.shape[2]
    TI = 512
    ni = N // TI
    return pl.pallas_call(
        _chamfer_kernel,
        grid=(B, ni),
        in_specs=[
            pl.BlockSpec((1, TI, 8), lambda b, i: (b, i, 0)),
            pl.BlockSpec((1, 8, M), lambda b, i: (b, 0, 0)),
        ],
        out_specs=pl.BlockSpec(memory_space=pltpu.SMEM),
        out_shape=jax.ShapeDtypeStruct((1, 1), jnp.float32),
        scratch_shapes=[
            pltpu.VMEM((1, M), jnp.float32),
            pltpu.SMEM((1, 1), jnp.float32),
        ],
    )(xp, yp)


def kernel(verts, target_points, target_normals):
    t = target_points
    v = verts
    B, N, _ = t.shape
    M = v.shape[1]
    xp = jnp.concatenate([t, jnp.zeros((B, N, 5), jnp.float32)], axis=-1)
    yp = jnp.concatenate([v, jnp.zeros((B, M, 5), jnp.float32)], axis=-1)
    yp = jnp.swapaxes(yp, 1, 2)
    out = _chamfer_pallas(xp, yp)
    return out[0, 0]


# speedup vs baseline: 2.2683x; 1.1531x over previous
import jax
import jax.numpy as jnp
from jax.experimental import pallas as pl
from jax.experimental.pallas import tpu as pltpu

_ALPHA = 0.3
_INV2A2 = 1.0 / (2.0 * _ALPHA * _ALPHA)


def _chamfer_kernel(x_ref, y_ref, out_ref, cmin_ref, acc_ref):
    b = pl.program_id(0)
    i = pl.program_id(1)
    nb = pl.num_programs(0)
    ni = pl.num_programs(1)

    @pl.when((b == 0) & (i == 0))
    def _init_acc():
        acc_ref[0, 0] = 0.0

    @pl.when(i == 0)
    def _init_cmin():
        cmin_ref[...] = jnp.full_like(cmin_ref, jnp.inf)

    x = x_ref[0]
    y = y_ref[0]
    xy = jnp.dot(x, y, preferred_element_type=jnp.float32)
    x2 = jnp.sum(x * x, axis=1, keepdims=True)
    y2 = 0.25 * jnp.sum(y * y, axis=0, keepdims=True)
    d = (x2 + y2) + xy

    rmin = jnp.min(d, axis=1)
    acc_ref[0, 0] += jnp.sum(jnp.exp(-(rmin * rmin) * _INV2A2) * rmin)

    cmin_ref[...] = jnp.minimum(cmin_ref[...], jnp.min(d, axis=0, keepdims=True))

    @pl.when(i == ni - 1)
    def _batch_end():
        c = cmin_ref[0]
        acc_ref[0, 0] += jnp.sum(jnp.exp(-(c * c) * _INV2A2) * c)

    @pl.when((b == nb - 1) & (i == ni - 1))
    def _final():
        out_ref[0, 0] = acc_ref[0, 0] / nb


def _chamfer_pallas(xp, yp):
    B, N, _ = xp.shape
    M = yp.shape[2]
    TI = 512
    ni = N // TI
    return pl.pallas_call(
        _chamfer_kernel,
        grid=(B, ni),
        in_specs=[
            pl.BlockSpec((1, TI, 8), lambda b, i: (b, i, 0)),
            pl.BlockSpec((1, 8, M), lambda b, i: (b, 0, 0)),
        ],
        out_specs=pl.BlockSpec(memory_space=pltpu.SMEM),
        out_shape=jax.ShapeDtypeStruct((1, 1), jnp.float32),
        scratch_shapes=[
            pltpu.VMEM((1, M), jnp.float32),
            pltpu.SMEM((1, 1), jnp.float32),
        ],
    )(xp, yp)


def kernel(verts, target_points, target_normals):
    t = target_points
    v = verts
    B, N, _ = t.shape
    M = v.shape[1]
    xp = jnp.concatenate([t, jnp.zeros((B, N, 5), jnp.float32)], axis=-1)
    yp = jnp.concatenate([-2.0 * v, jnp.zeros((B, M, 5), jnp.float32)], axis=-1)
    yp = jnp.swapaxes(yp, 1, 2)
    out = _chamfer_pallas(xp, yp)
    return out[0, 0]


# speedup vs baseline: 2.4230x; 1.0682x over previous
import jax
import jax.numpy as jnp
from jax.experimental import pallas as pl
from jax.experimental.pallas import tpu as pltpu

_ALPHA = 0.3
_INV2A2 = 1.0 / (2.0 * _ALPHA * _ALPHA)


def _chamfer_kernel(x_ref, y_ref, out_ref, cmin_ref, acc_ref):
    b = pl.program_id(0)
    i = pl.program_id(1)
    nb = pl.num_programs(0)
    ni = pl.num_programs(1)

    @pl.when((b == 0) & (i == 0))
    def _init_acc():
        acc_ref[0, 0] = 0.0

    @pl.when(i == 0)
    def _init_cmin():
        cmin_ref[...] = jnp.full_like(cmin_ref, jnp.inf)

    x = x_ref[0]
    y = y_ref[0]
    xy = jnp.dot(x, y, preferred_element_type=jnp.float32)
    x2 = jnp.sum(x * x, axis=1, keepdims=True)
    y2 = 0.25 * jnp.sum(y * y, axis=0, keepdims=True)
    d = (x2 + y2) + xy

    rmin = jnp.min(d, axis=1)
    acc_ref[0, 0] += jnp.sum(jnp.exp(-(rmin * rmin) * _INV2A2) * rmin)

    cmin_ref[...] = jnp.minimum(cmin_ref[...], jnp.min(d, axis=0, keepdims=True))

    @pl.when(i == ni - 1)
    def _batch_end():
        c = cmin_ref[0]
        acc_ref[0, 0] += jnp.sum(jnp.exp(-(c * c) * _INV2A2) * c)

    @pl.when((b == nb - 1) & (i == ni - 1))
    def _final():
        out_ref[0, 0] = acc_ref[0, 0] / nb


def _chamfer_pallas(xp, yp):
    B, N, _ = xp.shape
    M = yp.shape[2]
    TI = 1024
    ni = N // TI
    return pl.pallas_call(
        _chamfer_kernel,
        grid=(B, ni),
        in_specs=[
            pl.BlockSpec((1, TI, 8), lambda b, i: (b, i, 0)),
            pl.BlockSpec((1, 8, M), lambda b, i: (b, 0, 0)),
        ],
        out_specs=pl.BlockSpec(memory_space=pltpu.SMEM),
        out_shape=jax.ShapeDtypeStruct((1, 1), jnp.float32),
        scratch_shapes=[
            pltpu.VMEM((1, M), jnp.float32),
            pltpu.SMEM((1, 1), jnp.float32),
        ],
    )(xp, yp)


def kernel(verts, target_points, target_normals):
    t = target_points
    v = verts
    B, N, _ = t.shape
    M = v.shape[1]
    xp = jnp.concatenate([t, jnp.zeros((B, N, 5), jnp.float32)], axis=-1)
    yp = jnp.concatenate([-2.0 * v, jnp.zeros((B, M, 5), jnp.float32)], axis=-1)
    yp = jnp.swapaxes(yp, 1, 2)
    out = _chamfer_pallas(xp, yp)
    return out[0, 0]


# speedup vs baseline: 2.4886x; 1.0271x over previous
import jax
import jax.numpy as jnp
from jax.experimental import pallas as pl
from jax.experimental.pallas import tpu as pltpu

_ALPHA = 0.3
_INV2A2 = 1.0 / (2.0 * _ALPHA * _ALPHA)


def _chamfer_kernel(x_ref, y_ref, out_ref, cmin_ref, acc_ref):
    b = pl.program_id(0)
    i = pl.program_id(1)
    nb = pl.num_programs(0)
    ni = pl.num_programs(1)

    @pl.when((b == 0) & (i == 0))
    def _init_acc():
        acc_ref[0, 0] = 0.0

    @pl.when(i == 0)
    def _init_cmin():
        cmin_ref[...] = jnp.full_like(cmin_ref, jnp.inf)

    x = x_ref[0]
    y = y_ref[0]
    xy = jnp.dot(x, y, preferred_element_type=jnp.float32)
    x2 = jnp.sum(x * x, axis=1, keepdims=True)
    y2 = 0.25 * jnp.sum(y * y, axis=0, keepdims=True)
    d = (x2 + y2) + xy

    rmin = jnp.min(d, axis=1)
    acc_ref[0, 0] += jnp.sum(jnp.exp(-(rmin * rmin) * _INV2A2) * rmin)

    cmin_ref[...] = jnp.minimum(cmin_ref[...], jnp.min(d, axis=0, keepdims=True))

    @pl.when(i == ni - 1)
    def _batch_end():
        c = cmin_ref[0]
        acc_ref[0, 0] += jnp.sum(jnp.exp(-(c * c) * _INV2A2) * c)

    @pl.when((b == nb - 1) & (i == ni - 1))
    def _final():
        out_ref[0, 0] = acc_ref[0, 0] / nb


def _chamfer_pallas(xp, yp):
    B, N, _ = xp.shape
    M = yp.shape[2]
    TI = 2048
    ni = N // TI
    return pl.pallas_call(
        _chamfer_kernel,
        grid=(B, ni),
        in_specs=[
            pl.BlockSpec((1, TI, 8), lambda b, i: (b, i, 0)),
            pl.BlockSpec((1, 8, M), lambda b, i: (b, 0, 0)),
        ],
        out_specs=pl.BlockSpec(memory_space=pltpu.SMEM),
        out_shape=jax.ShapeDtypeStruct((1, 1), jnp.float32),
        scratch_shapes=[
            pltpu.VMEM((1, M), jnp.float32),
            pltpu.SMEM((1, 1), jnp.float32),
        ],
    )(xp, yp)


def kernel(verts, target_points, target_normals):
    t = target_points
    v = verts
    B, N, _ = t.shape
    M = v.shape[1]
    xp = jnp.concatenate([t, jnp.zeros((B, N, 5), jnp.float32)], axis=-1)
    yp = jnp.concatenate([-2.0 * v, jnp.zeros((B, M, 5), jnp.float32)], axis=-1)
    yp = jnp.swapaxes(yp, 1, 2)
    out = _chamfer_pallas(xp, yp)
    return out[0, 0]
